# baseline (speedup 1.0000x reference)
import functools

import numpy as np
import jax
import jax.numpy as jnp
from jax import lax
from jax.experimental import pallas as pl
from jax.experimental.pallas import tpu as pltpu
from jax.experimental.pallas import tpu_sc as plsc

N = 10000
E = 320000
CUTOFF = 11.5
NUM_RADIAL = 6
NUM_SPH = 3
NUM_POS_EMB = 16
HID = 128
HH = 64

NC = 2
NS = 16
NW = NC * NS
EP = 327680
EDGES_PER_TILE = EP // NW
CHUNK = 80
EPT_C = EP // NW
NCHUNK_C = EPT_C // CHUNK
GCHUNK = 2048
NGCHUNK = EDGES_PER_TILE // GCHUNK
NPAD = 10240
SROWS = 10112
SLAB = SROWS // NS
SLAB_CHUNKS = (128, 128, 128, 128, 120)
ZROWS = 128

NBLK = 1000
NBLK_A = 640
EBLK = 2048


def _embed_body(z_ref, bb_ref, sc_ref, esm_ref, wemb_ref, bemb_ref,
                wesm_ref, besm_ref, x_ref):
    z = z_ref[...]
    onehot = (lax.broadcasted_iota(jnp.int32, (NBLK_A, 26), 1) == z).astype(jnp.float32)
    xin = jnp.concatenate([onehot, bb_ref[...], sc_ref[...]], axis=1)
    lo = jnp.dot(xin, wemb_ref[...], preferred_element_type=jnp.float32) + bemb_ref[...]
    hi = jnp.dot(esm_ref[...], wesm_ref[...], preferred_element_type=jnp.float32) + besm_ref[...]
    x_ref[...] = jnp.concatenate([lo, hi], axis=1)


def _node_embed(z, bb, sc, esm, W_emb, b_emb, W_esm, b_esm):
    grid = NPAD // NBLK_A
    return pl.pallas_call(
        _embed_body,
        grid=(grid,),
        in_specs=[
            pl.BlockSpec((NBLK_A, 1), lambda n: (n, 0)),
            pl.BlockSpec((NBLK_A, 6), lambda n: (n, 0)),
            pl.BlockSpec((NBLK_A, 8), lambda n: (n, 0)),
            pl.BlockSpec((NBLK_A, 1280), lambda n: (n, 0)),
            pl.BlockSpec((40, 64), lambda n: (0, 0)),
            pl.BlockSpec((1, 64), lambda n: (0, 0)),
            pl.BlockSpec((1280, 64), lambda n: (0, 0)),
            pl.BlockSpec((1, 64), lambda n: (0, 0)),
        ],
        out_specs=pl.BlockSpec((NBLK_A, HID), lambda n: (n, 0)),
        out_shape=jax.ShapeDtypeStruct((NPAD, HID), jnp.float32),
    )(z, bb, sc, esm, W_emb, b_emb, W_esm, b_esm)


def _geom_body(px_hbm, py_hbm, pz_hbm, jidx_hbm, iidx_hbm, g8_hbm,
               px_v, py_v, pz_v, idxj_v, idxi_v, g8_v):
    c = lax.axis_index("c")
    s = lax.axis_index("s")
    pltpu.sync_copy(px_hbm, px_v)
    pltpu.sync_copy(py_hbm, py_v)
    pltpu.sync_copy(pz_hbm, pz_v)
    base = (c * NS + s) * EDGES_PER_TILE

    @pl.loop(0, NGCHUNK)
    def _chunk(t):
        off = base + t * GCHUNK
        pltpu.sync_copy(jidx_hbm.at[pl.ds(off, GCHUNK)], idxj_v)
        pltpu.sync_copy(iidx_hbm.at[pl.ds(off, GCHUNK)], idxi_v)

        @pl.loop(0, GCHUNK // 16)
        def _group(gr):
            sl = pl.ds(gr * 16, 16)
            q = gr // 8
            lo = (gr % 8) * 16
            slg = pl.ds(lo, 16)
            vi = idxi_v[sl]
            vj = idxj_v[sl]
            r0 = jnp.where(vi == 0, N - 1, vi - 1)
            r1 = jnp.where(vi == N - 1, 0, vi + 1)
            pjx = plsc.load_gather(px_v, [vj])
            pjy = plsc.load_gather(py_v, [vj])
            pjz = plsc.load_gather(pz_v, [vj])
            pix = plsc.load_gather(px_v, [vi])
            piy = plsc.load_gather(py_v, [vi])
            piz = plsc.load_gather(pz_v, [vi])
            p0x = plsc.load_gather(px_v, [r0])
            p0y = plsc.load_gather(py_v, [r0])
            p0z = plsc.load_gather(pz_v, [r0])
            p1x = plsc.load_gather(px_v, [r1])
            p1y = plsc.load_gather(py_v, [r1])
            p1z = plsc.load_gather(pz_v, [r1])
            jx = pjx - pix
            jy = pjy - piy
            jz = pjz - piz
            ax_ = p0x - pix
            ay_ = p0y - piy
            az_ = p0z - piz
            bx_ = p1x - pix
            by_ = p1y - piy
            bz_ = p1z - piz
            dist2 = jx * jx + jy * jy + jz * jz
            adot = jx * ax_ + jy * ay_ + jz * az_
            crx = jy * az_ - jz * ay_
            cry = jz * ax_ - jx * az_
            crz = jx * ay_ - jy * ax_
            bsq = crx * crx + cry * cry + crz * crz
            p1x_ = ay_ * bz_ - az_ * by_
            p1y_ = az_ * bx_ - ax_ * bz_
            p1z_ = ax_ * by_ - ay_ * bx_
            p2x_ = ay_ * jz - az_ * jy
            p2y_ = az_ * jx - ax_ * jz
            p2z_ = ax_ * jy - ay_ * jx
            a2 = p1x_ * p2x_ + p1y_ * p2y_ + p1z_ * p2z_
            ccx = p1y_ * p2z_ - p1z_ * p2y_
            ccy = p1z_ * p2x_ - p1x_ * p2z_
            ccz = p1x_ * p2y_ - p1y_ * p2x_
            tval = ccx * ax_ + ccy * ay_ + ccz * az_
            r0sq = ax_ * ax_ + ay_ * ay_ + az_ * az_
            dm = (vj - vi).astype(jnp.float32)
            g8_v[0, q, slg] = dist2
            g8_v[1, q, slg] = adot
            g8_v[2, q, slg] = bsq
            g8_v[3, q, slg] = a2
            g8_v[4, q, slg] = tval
            g8_v[5, q, slg] = r0sq
            g8_v[6, q, slg] = dm
            g8_v[7, q, slg] = dm

        row = pl.multiple_of(off // 128, 16)
        pltpu.sync_copy(g8_v, g8_hbm.at[:, pl.ds(row, GCHUNK // 128)])


def _sc_geom(px, py, pz, jidx, iidx):
    mesh = plsc.VectorSubcoreMesh(core_axis_name="c", subcore_axis_name="s",
                                  num_cores=NC, num_subcores=NS)
    f = pl.kernel(
        _geom_body,
        out_type=jax.ShapeDtypeStruct((8, EP // 128, 128), jnp.float32),
        mesh=mesh,
        compiler_params=pltpu.CompilerParams(needs_layout_passes=False),
        scratch_types=[
            pltpu.VMEM((NPAD,), jnp.float32),
            pltpu.VMEM((NPAD,), jnp.float32),
            pltpu.VMEM((NPAD,), jnp.float32),
            pltpu.VMEM((GCHUNK,), jnp.int32),
            pltpu.VMEM((GCHUNK,), jnp.int32),
            pltpu.VMEM((8, GCHUNK // 128, 128), jnp.float32),
        ],
    )
    return f(px, py, pz, jidx, iidx)


RB = EBLK // 128


def _edge_body(g3_ref, wall_ref, act_ref):
    g3 = g3_ref[...]
    dist2 = g3[0]
    a = g3[1]
    bsq = g3[2]
    a2 = g3[3]
    t = g3[4]
    r0sq = g3[5]
    dm = g3[6]

    d = jnp.maximum(jnp.sqrt(dist2), 1e-6)
    dn = a * a + bsq
    ct = jnp.where(dn == 0.0, 1.0, a * lax.rsqrt(dn))
    c2t = 2.0 * ct * ct - 1.0
    b2 = t / (jnp.sqrt(r0sq) + 1e-9)
    qn = a2 * a2 + b2 * b2
    cp = jnp.where(qn == 0.0, 1.0, a2 * lax.rsqrt(qn))
    c2p = 2.0 * cp * cp - 1.0

    x = d * (np.pi / CUTOFF)
    s1 = jnp.sin(x)
    c1 = jnp.cos(x)
    env = jnp.exp(-(d / CUTOFF) ** 2) * np.sqrt(2.0 / CUTOFF) / d
    two_c1 = 2.0 * c1
    sl = [s1, two_c1 * s1]
    for _ in range(4):
        sl.append(two_c1 * sl[-1] - sl[-2])
    rbf = [sn * env for sn in sl]
    angt = [None, ct, c2t]
    angp = [None, cp, c2p]

    rows = []
    for n in range(6):
        for l1 in range(3):
            for l2 in range(3):
                v = rbf[n]
                if angt[l1] is not None:
                    v = v * angt[l1]
                if angp[l2] is not None:
                    v = v * angp[l2]
                rows.append(v)
    for k in range(8):
        fk = float(np.exp(-2.0 * k * np.log(10000.0) / NUM_POS_EMB))
        rows.append(jnp.cos(dm * fk))
    for k in range(8):
        fk = float(np.exp(-2.0 * k * np.log(10000.0) / NUM_POS_EMB))
        rows.append(jnp.sin(dm * fk))

    P = jnp.stack(rows, axis=0).reshape(70, EBLK)
    u = lax.dot_general(P, wall_ref[...],
                        dimension_numbers=(((0,), (0,)), ((), ())),
                        preferred_element_type=jnp.float32)
    act_ref[...] = u * jax.nn.sigmoid(u)


def _edge_act(g3, W_all):
    grid = EP // EBLK
    return pl.pallas_call(
        _edge_body,
        grid=(grid,),
        in_specs=[
            pl.BlockSpec((8, RB, 128), lambda e: (0, e, 0)),
            pl.BlockSpec((70, HID), lambda e: (0, 0)),
        ],
        out_specs=pl.BlockSpec((EBLK, HID), lambda e: (e, 0)),
        out_shape=jax.ShapeDtypeStruct((EP, HID), jnp.float32),
    )(g3, W_all)


def _sc_body(x_hbm, act_hbm, jidx_hbm, iidx_hbm, out_hbm,
             jx, ix, xr, ab, zbuf_v, hacc, gsem):
    c = lax.axis_index("c")
    s = lax.axis_index("s")

    zeros16 = jnp.zeros((16,), jnp.float32)

    @pl.loop(0, 8)
    def _zero(r):
        for k in range(HID // 16):
            zbuf_v[r, pl.ds(k * 16, 16)] = zeros16

    row0 = s * SLAB

    @pl.loop(0, SLAB // 8)
    def _zslab(r):
        pltpu.sync_copy(zbuf_v, hacc.at[pl.ds(row0 + r * 8, 8)])

    plsc.subcore_barrier()

    w = c * NS + s
    ebase = w * EPT_C

    @pl.loop(0, NCHUNK_C + 2)
    def _pipe(t):
        b = t % 2

        @pl.when(t >= 2)
        def _consume():
            tc = t - 2
            off = pl.multiple_of(ebase + tc * CHUNK, CHUNK)
            pltpu.sync_copy(act_hbm.at[pl.ds(off, CHUNK)], ab.at[b])
            pltpu.make_async_copy(x_hbm.at[jx.at[b]], xr.at[b],
                                  gsem.at[b]).wait()

            @pl.loop(0, CHUNK, unroll=4)
            def _mul(r):
                for k in range(HID // 16):
                    slk = pl.ds(k * 16, 16)
                    ab[b, r, slk] = ab[b, r, slk] * xr[b, r, slk]

            pltpu.sync_copy(ab.at[b], hacc.at[ix.at[b]], add=True)

        @pl.when(t < NCHUNK_C)
        def _start():
            off = pl.multiple_of(ebase + t * CHUNK, CHUNK)
            pltpu.sync_copy(jidx_hbm.at[pl.ds(off, CHUNK)], jx.at[b])
            pltpu.sync_copy(iidx_hbm.at[pl.ds(off, CHUNK)], ix.at[b])
            pltpu.async_copy(x_hbm.at[jx.at[b]], xr.at[b], gsem.at[b])

    plsc.subcore_barrier()
    pltpu.sync_copy(hacc.at[pl.ds(row0, SLAB)],
                    out_hbm.at[c, pl.ds(row0, SLAB)])


def _sc_gather_scatter(x, act, jidx, iidx):
    mesh = plsc.VectorSubcoreMesh(core_axis_name="c", subcore_axis_name="s",
                                  num_cores=NC, num_subcores=NS)
    f = pl.kernel(
        _sc_body,
        out_type=jax.ShapeDtypeStruct((NC, SROWS, HID), jnp.float32),
        mesh=mesh,
        scratch_types=[
            pltpu.VMEM((2, CHUNK), jnp.int32),
            pltpu.VMEM((2, CHUNK), jnp.int32),
            pltpu.VMEM((2, CHUNK, HID), jnp.float32),
            pltpu.VMEM((2, CHUNK, HID), jnp.float32),
            pltpu.VMEM((8, HID), jnp.float32),
            pltpu.VMEM_SHARED((SROWS, HID), jnp.float32),
            pltpu.SemaphoreType.DMA((2,)),
        ],
    )
    return f(x, act, jidx, iidx)


def _final_body(x_ref, h0_ref, h1_ref, w1_ref, b1_ref, w2_ref, b2_ref,
                wl_ref, bl_ref, wn1_ref, bn1_ref, wno_ref, bno_ref, out_ref):
    h = x_ref[...] + h0_ref[0] + h1_ref[0]
    o = jax.nn.relu(jnp.dot(h, w1_ref[...], preferred_element_type=jnp.float32) + b1_ref[...])
    o = jax.nn.relu(jnp.dot(o, w2_ref[...], preferred_element_type=jnp.float32) + b2_ref[...])
    site = jax.nn.sigmoid(jnp.dot(o, wl_ref[...], preferred_element_type=jnp.float32) + bl_ref[...])
    n1 = jax.nn.relu(jnp.dot(h, wn1_ref[...], preferred_element_type=jnp.float32) + bn1_ref[...])
    node = jnp.dot(n1, wno_ref[...], preferred_element_type=jnp.float32) + bno_ref[...]
    out_ref[...] = jnp.concatenate([site, node], axis=1)


def _final(x, hp, W_out1, b_out1, W_out2, b_out2, W_lin_out, b_lin_out,
           W_node1, b_node1, W_node_out, b_node_out):
    grid = N // NBLK
    return pl.pallas_call(
        _final_body,
        grid=(grid,),
        in_specs=[
            pl.BlockSpec((NBLK, HID), lambda n: (n, 0)),
            pl.BlockSpec((1, NBLK, HID), lambda n: (0, n, 0)),
            pl.BlockSpec((1, NBLK, HID), lambda n: (1, n, 0)),
            pl.BlockSpec((HID, HID), lambda n: (0, 0)),
            pl.BlockSpec((1, HID), lambda n: (0, 0)),
            pl.BlockSpec((HID, 32), lambda n: (0, 0)),
            pl.BlockSpec((1, 32), lambda n: (0, 0)),
            pl.BlockSpec((32, 1), lambda n: (0, 0)),
            pl.BlockSpec((1, 1), lambda n: (0, 0)),
            pl.BlockSpec((HID, 32), lambda n: (0, 0)),
            pl.BlockSpec((1, 32), lambda n: (0, 0)),
            pl.BlockSpec((32, 2), lambda n: (0, 0)),
            pl.BlockSpec((1, 2), lambda n: (0, 0)),
        ],
        out_specs=pl.BlockSpec((NBLK, 3), lambda n: (n, 0)),
        out_shape=jax.ShapeDtypeStruct((N, 3), jnp.float32),
    )(x, hp, hp, W_out1, b_out1, W_out2, b_out2, W_lin_out, b_lin_out,
      W_node1, b_node1, W_node_out, b_node_out)


def kernel(coords_ca, coords_n, coords_c, bb_embs, side_chain_embs, esm_emb,
           W_emb, b_emb, W_esm, b_esm, W_msg, W_pe, W_out1, b_out1, W_out2,
           b_out2, W_lin_out, b_lin_out, W_node1, b_node1, W_node_out,
           b_node_out, z, edge_index, batch):
    del coords_n, coords_c, batch
    npad = NPAD - N
    z2 = jnp.concatenate([z.astype(jnp.int32),
                          jnp.zeros((npad,), jnp.int32)]).reshape(NPAD, 1)
    bb_p = jnp.concatenate([bb_embs, jnp.zeros((npad, 6), jnp.float32)])
    sc_p = jnp.concatenate([side_chain_embs, jnp.zeros((npad, 8), jnp.float32)])
    esm_p = jnp.concatenate([esm_emb, jnp.zeros((npad, 1280), jnp.float32)])
    x = _node_embed(z2, bb_p, sc_p, esm_p,
                    W_emb, b_emb.reshape(1, -1), W_esm,
                    b_esm.reshape(1, -1))

    jidx = jnp.concatenate([edge_index[0].astype(jnp.int32),
                            jnp.zeros((EP - E,), jnp.int32)])
    iidx = jnp.concatenate([edge_index[1].astype(jnp.int32),
                            jnp.full((EP - E,), N, jnp.int32)])
    zpad = jnp.zeros((NPAD - N,), jnp.float32)
    px = jnp.concatenate([coords_ca[:, 0], zpad])
    py = jnp.concatenate([coords_ca[:, 1], zpad])
    pz = jnp.concatenate([coords_ca[:, 2], zpad])
    g8 = _sc_geom(px, py, pz, jidx, iidx)
    W_all = jnp.concatenate([W_msg, W_pe], axis=0)
    act = _edge_act(g8, W_all)
    hp = _sc_gather_scatter(x, act, jidx, iidx)
    return _final(x, hp, W_out1, b_out1.reshape(1, -1), W_out2,
                  b_out2.reshape(1, -1), W_lin_out, b_lin_out.reshape(1, -1),
                  W_node1, b_node1.reshape(1, -1), W_node_out,
                  b_node_out.reshape(1, -1))


# speedup vs baseline: 1.0406x; 1.0406x over previous
import functools

import numpy as np
import jax
import jax.numpy as jnp
from jax import lax
from jax.experimental import pallas as pl
from jax.experimental.pallas import tpu as pltpu
from jax.experimental.pallas import tpu_sc as plsc

N = 10000
E = 320000
CUTOFF = 11.5
NUM_RADIAL = 6
NUM_SPH = 3
NUM_POS_EMB = 16
HID = 128
HH = 64

NC = 2
NS = 16
NW = NC * NS
EP = 327680
EDGES_PER_TILE = EP // NW
CHUNK = 80
EPT_C = EP // NW
NCHUNK_C = EPT_C // CHUNK
GCHUNK = 2048
NGCHUNK = EDGES_PER_TILE // GCHUNK
NPAD = 10240
SROWS = 10112
SLAB = SROWS // NS
SLAB_CHUNKS = (128, 128, 128, 128, 120)
ZROWS = 128

NBLK = 1000
NBLK_A = 640
EBLK = 2048


def _embed_body(z_ref, bb_ref, sc_ref, esm_ref, wemb_ref, bemb_ref,
                wesm_ref, besm_ref, x_ref):
    z = z_ref[...]
    onehot = (lax.broadcasted_iota(jnp.int32, (NBLK_A, 26), 1) == z).astype(jnp.float32)
    xin = jnp.concatenate([onehot, bb_ref[...], sc_ref[...]], axis=1)
    lo = jnp.dot(xin, wemb_ref[...], preferred_element_type=jnp.float32) + bemb_ref[...]
    hi = jnp.dot(esm_ref[...], wesm_ref[...], preferred_element_type=jnp.float32) + besm_ref[...]
    x_ref[...] = jnp.concatenate([lo, hi], axis=1)


def _node_embed(z, bb, sc, esm, W_emb, b_emb, W_esm, b_esm):
    grid = NPAD // NBLK_A
    return pl.pallas_call(
        _embed_body,
        grid=(grid,),
        in_specs=[
            pl.BlockSpec((NBLK_A, 1), lambda n: (n, 0)),
            pl.BlockSpec((NBLK_A, 6), lambda n: (n, 0)),
            pl.BlockSpec((NBLK_A, 8), lambda n: (n, 0)),
            pl.BlockSpec((NBLK_A, 1280), lambda n: (n, 0)),
            pl.BlockSpec((40, 64), lambda n: (0, 0)),
            pl.BlockSpec((1, 64), lambda n: (0, 0)),
            pl.BlockSpec((1280, 64), lambda n: (0, 0)),
            pl.BlockSpec((1, 64), lambda n: (0, 0)),
        ],
        out_specs=pl.BlockSpec((NBLK_A, HID), lambda n: (n, 0)),
        out_shape=jax.ShapeDtypeStruct((NPAD, HID), jnp.float32),
    )(z, bb, sc, esm, W_emb, b_emb, W_esm, b_esm)


def _geom_body(px_hbm, py_hbm, pz_hbm, jidx_hbm, iidx_hbm, g8_hbm,
               px_v, py_v, pz_v, idxj_v, idxi_v, g8_v):
    c = lax.axis_index("c")
    s = lax.axis_index("s")
    pltpu.sync_copy(px_hbm, px_v)
    pltpu.sync_copy(py_hbm, py_v)
    pltpu.sync_copy(pz_hbm, pz_v)
    base = (c * NS + s) * EDGES_PER_TILE

    @pl.loop(0, NGCHUNK)
    def _chunk(t):
        off = base + t * GCHUNK
        pltpu.sync_copy(jidx_hbm.at[pl.ds(off, GCHUNK)], idxj_v)
        pltpu.sync_copy(iidx_hbm.at[pl.ds(off, GCHUNK)], idxi_v)

        @pl.loop(0, GCHUNK // 16)
        def _group(gr):
            sl = pl.ds(gr * 16, 16)
            q = gr // 8
            lo = (gr % 8) * 16
            slg = pl.ds(lo, 16)
            vi = idxi_v[sl]
            vj = idxj_v[sl]
            r0 = jnp.where(vi == 0, N - 1, vi - 1)
            r1 = jnp.where(vi == N - 1, 0, vi + 1)
            pjx = plsc.load_gather(px_v, [vj])
            pjy = plsc.load_gather(py_v, [vj])
            pjz = plsc.load_gather(pz_v, [vj])
            pix = plsc.load_gather(px_v, [vi])
            piy = plsc.load_gather(py_v, [vi])
            piz = plsc.load_gather(pz_v, [vi])
            p0x = plsc.load_gather(px_v, [r0])
            p0y = plsc.load_gather(py_v, [r0])
            p0z = plsc.load_gather(pz_v, [r0])
            p1x = plsc.load_gather(px_v, [r1])
            p1y = plsc.load_gather(py_v, [r1])
            p1z = plsc.load_gather(pz_v, [r1])
            jx = pjx - pix
            jy = pjy - piy
            jz = pjz - piz
            ax_ = p0x - pix
            ay_ = p0y - piy
            az_ = p0z - piz
            bx_ = p1x - pix
            by_ = p1y - piy
            bz_ = p1z - piz
            dist2 = jx * jx + jy * jy + jz * jz
            adot = jx * ax_ + jy * ay_ + jz * az_
            crx = jy * az_ - jz * ay_
            cry = jz * ax_ - jx * az_
            crz = jx * ay_ - jy * ax_
            bsq = crx * crx + cry * cry + crz * crz
            p1x_ = ay_ * bz_ - az_ * by_
            p1y_ = az_ * bx_ - ax_ * bz_
            p1z_ = ax_ * by_ - ay_ * bx_
            p2x_ = ay_ * jz - az_ * jy
            p2y_ = az_ * jx - ax_ * jz
            p2z_ = ax_ * jy - ay_ * jx
            a2 = p1x_ * p2x_ + p1y_ * p2y_ + p1z_ * p2z_
            ccx = p1y_ * p2z_ - p1z_ * p2y_
            ccy = p1z_ * p2x_ - p1x_ * p2z_
            ccz = p1x_ * p2y_ - p1y_ * p2x_
            tval = ccx * ax_ + ccy * ay_ + ccz * az_
            r0sq = ax_ * ax_ + ay_ * ay_ + az_ * az_
            dm = (vj - vi).astype(jnp.float32)
            g8_v[0, q, slg] = dist2
            g8_v[1, q, slg] = adot
            g8_v[2, q, slg] = bsq
            g8_v[3, q, slg] = a2
            g8_v[4, q, slg] = tval
            g8_v[5, q, slg] = r0sq
            g8_v[6, q, slg] = dm
            g8_v[7, q, slg] = dm

        row = pl.multiple_of(off // 128, 16)
        pltpu.sync_copy(g8_v, g8_hbm.at[:, pl.ds(row, GCHUNK // 128)])


def _sc_geom(px, py, pz, jidx, iidx):
    mesh = plsc.VectorSubcoreMesh(core_axis_name="c", subcore_axis_name="s",
                                  num_cores=NC, num_subcores=NS)
    f = pl.kernel(
        _geom_body,
        out_type=jax.ShapeDtypeStruct((8, EP // 128, 128), jnp.float32),
        mesh=mesh,
        compiler_params=pltpu.CompilerParams(needs_layout_passes=False),
        scratch_types=[
            pltpu.VMEM((NPAD,), jnp.float32),
            pltpu.VMEM((NPAD,), jnp.float32),
            pltpu.VMEM((NPAD,), jnp.float32),
            pltpu.VMEM((GCHUNK,), jnp.int32),
            pltpu.VMEM((GCHUNK,), jnp.int32),
            pltpu.VMEM((8, GCHUNK // 128, 128), jnp.float32),
        ],
    )
    return f(px, py, pz, jidx, iidx)


RB = EBLK // 128


def _edge_body(g3_ref, wall_ref, act_ref):
    g3 = g3_ref[...]
    dist2 = g3[0]
    a = g3[1]
    bsq = g3[2]
    a2 = g3[3]
    t = g3[4]
    r0sq = g3[5]
    dm = g3[6]

    d = jnp.maximum(jnp.sqrt(dist2), 1e-6)
    dn = a * a + bsq
    ct = jnp.where(dn == 0.0, 1.0, a * lax.rsqrt(dn))
    c2t = 2.0 * ct * ct - 1.0
    b2 = t / (jnp.sqrt(r0sq) + 1e-9)
    qn = a2 * a2 + b2 * b2
    cp = jnp.where(qn == 0.0, 1.0, a2 * lax.rsqrt(qn))
    c2p = 2.0 * cp * cp - 1.0

    x = d * (np.pi / CUTOFF)
    s1 = jnp.sin(x)
    c1 = jnp.cos(x)
    env = jnp.exp(-(d / CUTOFF) ** 2) * np.sqrt(2.0 / CUTOFF) / d
    two_c1 = 2.0 * c1
    sl = [s1, two_c1 * s1]
    for _ in range(4):
        sl.append(two_c1 * sl[-1] - sl[-2])
    rbf = [sn * env for sn in sl]
    angt = [None, ct, c2t]
    angp = [None, cp, c2p]

    rows = []
    for n in range(6):
        for l1 in range(3):
            for l2 in range(3):
                v = rbf[n]
                if angt[l1] is not None:
                    v = v * angt[l1]
                if angp[l2] is not None:
                    v = v * angp[l2]
                rows.append(v)
    for k in range(8):
        fk = float(np.exp(-2.0 * k * np.log(10000.0) / NUM_POS_EMB))
        rows.append(jnp.cos(dm * fk))
    for k in range(8):
        fk = float(np.exp(-2.0 * k * np.log(10000.0) / NUM_POS_EMB))
        rows.append(jnp.sin(dm * fk))

    P = jnp.stack(rows, axis=0).reshape(70, EBLK)
    u = lax.dot_general(P, wall_ref[...],
                        dimension_numbers=(((0,), (0,)), ((), ())),
                        preferred_element_type=jnp.float32)
    act_ref[...] = u * jax.nn.sigmoid(u)


def _edge_act(g3, W_all):
    grid = EP // EBLK
    return pl.pallas_call(
        _edge_body,
        grid=(grid,),
        in_specs=[
            pl.BlockSpec((8, RB, 128), lambda e: (0, e, 0)),
            pl.BlockSpec((70, HID), lambda e: (0, 0)),
        ],
        out_specs=pl.BlockSpec((EBLK, HID), lambda e: (e, 0)),
        out_shape=jax.ShapeDtypeStruct((EP, HID), jnp.float32),
    )(g3, W_all)


def _sc_body(x_hbm, act_hbm, jidx_hbm, iidx_hbm, out_hbm,
             jx, ix, xr, ab, zbuf_v, hacc, gsem, ssem):
    c = lax.axis_index("c")
    s = lax.axis_index("s")

    zeros16 = jnp.zeros((16,), jnp.float32)

    @pl.loop(0, 8)
    def _zero(r):
        for k in range(HID // 16):
            zbuf_v[r, pl.ds(k * 16, 16)] = zeros16

    row0 = s * SLAB

    @pl.loop(0, SLAB // 8)
    def _zslab(r):
        pltpu.sync_copy(zbuf_v, hacc.at[pl.ds(row0 + r * 8, 8)])

    plsc.subcore_barrier()

    w = c * NS + s
    ebase = w * EPT_C

    @pl.loop(0, NCHUNK_C + 4)
    def _pipe(t):
        b = t % 2

        @pl.when(t >= 4)
        def _drain():
            pltpu.make_async_copy(ab.at[b], hacc.at[ix.at[(t - 4) % 4]],
                                  ssem.at[b]).wait()

        @pl.when((t >= 2) & (t < NCHUNK_C + 2))
        def _consume():
            tc = t - 2
            off = pl.multiple_of(ebase + tc * CHUNK, CHUNK)
            pltpu.sync_copy(act_hbm.at[pl.ds(off, CHUNK)], ab.at[b])
            pltpu.make_async_copy(x_hbm.at[jx.at[b]], xr.at[b],
                                  gsem.at[b]).wait()

            @pl.loop(0, CHUNK, unroll=4)
            def _mul(r):
                for k in range(HID // 16):
                    slk = pl.ds(k * 16, 16)
                    ab[b, r, slk] = ab[b, r, slk] * xr[b, r, slk]

            pltpu.async_copy(ab.at[b], hacc.at[ix.at[tc % 4]], ssem.at[b],
                             add=True)

        @pl.when(t < NCHUNK_C)
        def _start():
            off = pl.multiple_of(ebase + t * CHUNK, CHUNK)
            pltpu.sync_copy(jidx_hbm.at[pl.ds(off, CHUNK)], jx.at[b])
            pltpu.sync_copy(iidx_hbm.at[pl.ds(off, CHUNK)], ix.at[t % 4])
            pltpu.async_copy(x_hbm.at[jx.at[b]], xr.at[b], gsem.at[b])

    plsc.subcore_barrier()
    pltpu.sync_copy(hacc.at[pl.ds(row0, SLAB)],
                    out_hbm.at[c, pl.ds(row0, SLAB)])


def _sc_gather_scatter(x, act, jidx, iidx):
    mesh = plsc.VectorSubcoreMesh(core_axis_name="c", subcore_axis_name="s",
                                  num_cores=NC, num_subcores=NS)
    f = pl.kernel(
        _sc_body,
        out_type=jax.ShapeDtypeStruct((NC, SROWS, HID), jnp.float32),
        mesh=mesh,
        scratch_types=[
            pltpu.VMEM((2, CHUNK), jnp.int32),
            pltpu.VMEM((4, CHUNK), jnp.int32),
            pltpu.VMEM((2, CHUNK, HID), jnp.float32),
            pltpu.VMEM((2, CHUNK, HID), jnp.float32),
            pltpu.VMEM((8, HID), jnp.float32),
            pltpu.VMEM_SHARED((SROWS, HID), jnp.float32),
            pltpu.SemaphoreType.DMA((2,)),
            pltpu.SemaphoreType.DMA((2,)),
        ],
    )
    return f(x, act, jidx, iidx)


def _final_body(x_ref, h0_ref, h1_ref, w1_ref, b1_ref, w2_ref, b2_ref,
                wl_ref, bl_ref, wn1_ref, bn1_ref, wno_ref, bno_ref, out_ref):
    h = x_ref[...] + h0_ref[0] + h1_ref[0]
    o = jax.nn.relu(jnp.dot(h, w1_ref[...], preferred_element_type=jnp.float32) + b1_ref[...])
    o = jax.nn.relu(jnp.dot(o, w2_ref[...], preferred_element_type=jnp.float32) + b2_ref[...])
    site = jax.nn.sigmoid(jnp.dot(o, wl_ref[...], preferred_element_type=jnp.float32) + bl_ref[...])
    n1 = jax.nn.relu(jnp.dot(h, wn1_ref[...], preferred_element_type=jnp.float32) + bn1_ref[...])
    node = jnp.dot(n1, wno_ref[...], preferred_element_type=jnp.float32) + bno_ref[...]
    out_ref[...] = jnp.concatenate([site, node], axis=1)


def _final(x, hp, W_out1, b_out1, W_out2, b_out2, W_lin_out, b_lin_out,
           W_node1, b_node1, W_node_out, b_node_out):
    grid = N // NBLK
    return pl.pallas_call(
        _final_body,
        grid=(grid,),
        in_specs=[
            pl.BlockSpec((NBLK, HID), lambda n: (n, 0)),
            pl.BlockSpec((1, NBLK, HID), lambda n: (0, n, 0)),
            pl.BlockSpec((1, NBLK, HID), lambda n: (1, n, 0)),
            pl.BlockSpec((HID, HID), lambda n: (0, 0)),
            pl.BlockSpec((1, HID), lambda n: (0, 0)),
            pl.BlockSpec((HID, 32), lambda n: (0, 0)),
            pl.BlockSpec((1, 32), lambda n: (0, 0)),
            pl.BlockSpec((32, 1), lambda n: (0, 0)),
            pl.BlockSpec((1, 1), lambda n: (0, 0)),
            pl.BlockSpec((HID, 32), lambda n: (0, 0)),
            pl.BlockSpec((1, 32), lambda n: (0, 0)),
            pl.BlockSpec((32, 2), lambda n: (0, 0)),
            pl.BlockSpec((1, 2), lambda n: (0, 0)),
        ],
        out_specs=pl.BlockSpec((NBLK, 3), lambda n: (n, 0)),
        out_shape=jax.ShapeDtypeStruct((N, 3), jnp.float32),
    )(x, hp, hp, W_out1, b_out1, W_out2, b_out2, W_lin_out, b_lin_out,
      W_node1, b_node1, W_node_out, b_node_out)


def kernel(coords_ca, coords_n, coords_c, bb_embs, side_chain_embs, esm_emb,
           W_emb, b_emb, W_esm, b_esm, W_msg, W_pe, W_out1, b_out1, W_out2,
           b_out2, W_lin_out, b_lin_out, W_node1, b_node1, W_node_out,
           b_node_out, z, edge_index, batch):
    del coords_n, coords_c, batch
    npad = NPAD - N
    z2 = jnp.concatenate([z.astype(jnp.int32),
                          jnp.zeros((npad,), jnp.int32)]).reshape(NPAD, 1)
    bb_p = jnp.concatenate([bb_embs, jnp.zeros((npad, 6), jnp.float32)])
    sc_p = jnp.concatenate([side_chain_embs, jnp.zeros((npad, 8), jnp.float32)])
    esm_p = jnp.concatenate([esm_emb, jnp.zeros((npad, 1280), jnp.float32)])
    x = _node_embed(z2, bb_p, sc_p, esm_p,
                    W_emb, b_emb.reshape(1, -1), W_esm,
                    b_esm.reshape(1, -1))

    jidx = jnp.concatenate([edge_index[0].astype(jnp.int32),
                            jnp.zeros((EP - E,), jnp.int32)])
    iidx = jnp.concatenate([edge_index[1].astype(jnp.int32),
                            jnp.full((EP - E,), N, jnp.int32)])
    zpad = jnp.zeros((NPAD - N,), jnp.float32)
    px = jnp.concatenate([coords_ca[:, 0], zpad])
    py = jnp.concatenate([coords_ca[:, 1], zpad])
    pz = jnp.concatenate([coords_ca[:, 2], zpad])
    g8 = _sc_geom(px, py, pz, jidx, iidx)
    W_all = jnp.concatenate([W_msg, W_pe], axis=0)
    act = _edge_act(g8, W_all)
    hp = _sc_gather_scatter(x, act, jidx, iidx)
    return _final(x, hp, W_out1, b_out1.reshape(1, -1), W_out2,
                  b_out2.reshape(1, -1), W_lin_out, b_lin_out.reshape(1, -1),
                  W_node1, b_node1.reshape(1, -1), W_node_out,
                  b_node_out.reshape(1, -1))


# speedup vs baseline: 1.0443x; 1.0036x over previous
import functools

import numpy as np
import jax
import jax.numpy as jnp
from jax import lax
from jax.experimental import pallas as pl
from jax.experimental.pallas import tpu as pltpu
from jax.experimental.pallas import tpu_sc as plsc

N = 10000
E = 320000
CUTOFF = 11.5
NUM_RADIAL = 6
NUM_SPH = 3
NUM_POS_EMB = 16
HID = 128
HH = 64

NC = 2
NS = 16
NW = NC * NS
EP = 327680
EDGES_PER_TILE = EP // NW
CHUNK = 80
EPT_C = EP // NW
NCHUNK_C = EPT_C // CHUNK
GCHUNK = 2048
NGCHUNK = EDGES_PER_TILE // GCHUNK
NPAD = 10240
SROWS = 10240
SLAB = SROWS // NS

NBLK = 1000
NBLK_A = 640
EBLK = 2048


def _embed_body(z_ref, bb_ref, sc_ref, esm_ref, wemb_ref, bemb_ref,
                wesm_ref, besm_ref, x_ref):
    z = z_ref[...]
    onehot = (lax.broadcasted_iota(jnp.int32, (NBLK_A, 26), 1) == z).astype(jnp.float32)
    xin = jnp.concatenate([onehot, bb_ref[...], sc_ref[...]], axis=1)
    lo = jnp.dot(xin, wemb_ref[...], preferred_element_type=jnp.float32) + bemb_ref[...]
    hi = jnp.dot(esm_ref[...], wesm_ref[...], preferred_element_type=jnp.float32) + besm_ref[...]
    x_ref[...] = jnp.concatenate([lo, hi], axis=1)


def _node_embed(z, bb, sc, esm, W_emb, b_emb, W_esm, b_esm):
    grid = NPAD // NBLK_A
    return pl.pallas_call(
        _embed_body,
        grid=(grid,),
        in_specs=[
            pl.BlockSpec((NBLK_A, 1), lambda n: (n, 0)),
            pl.BlockSpec((NBLK_A, 6), lambda n: (n, 0)),
            pl.BlockSpec((NBLK_A, 8), lambda n: (n, 0)),
            pl.BlockSpec((NBLK_A, 1280), lambda n: (n, 0)),
            pl.BlockSpec((40, 64), lambda n: (0, 0)),
            pl.BlockSpec((1, 64), lambda n: (0, 0)),
            pl.BlockSpec((1280, 64), lambda n: (0, 0)),
            pl.BlockSpec((1, 64), lambda n: (0, 0)),
        ],
        out_specs=pl.BlockSpec((NBLK_A, HID), lambda n: (n, 0)),
        out_shape=jax.ShapeDtypeStruct((NPAD, HID), jnp.float32),
    )(z, bb, sc, esm, W_emb, b_emb, W_esm, b_esm)


def _geom_body(px_hbm, py_hbm, pz_hbm, jidx_hbm, iidx_hbm, g8_hbm,
               px_v, py_v, pz_v, idxj_v, idxi_v, g8_v):
    c = lax.axis_index("c")
    s = lax.axis_index("s")
    pltpu.sync_copy(px_hbm, px_v)
    pltpu.sync_copy(py_hbm, py_v)
    pltpu.sync_copy(pz_hbm, pz_v)
    base = (c * NS + s) * EDGES_PER_TILE

    @pl.loop(0, NGCHUNK)
    def _chunk(t):
        off = base + t * GCHUNK
        pltpu.sync_copy(jidx_hbm.at[pl.ds(off, GCHUNK)], idxj_v)
        pltpu.sync_copy(iidx_hbm.at[pl.ds(off, GCHUNK)], idxi_v)

        @pl.loop(0, GCHUNK // 16)
        def _group(gr):
            sl = pl.ds(gr * 16, 16)
            q = gr // 8
            lo = (gr % 8) * 16
            slg = pl.ds(lo, 16)
            vi = idxi_v[sl]
            vj = idxj_v[sl]
            r0 = jnp.where(vi == 0, N - 1, vi - 1)
            r1 = jnp.where(vi == N - 1, 0, vi + 1)
            pjx = plsc.load_gather(px_v, [vj])
            pjy = plsc.load_gather(py_v, [vj])
            pjz = plsc.load_gather(pz_v, [vj])
            pix = plsc.load_gather(px_v, [vi])
            piy = plsc.load_gather(py_v, [vi])
            piz = plsc.load_gather(pz_v, [vi])
            p0x = plsc.load_gather(px_v, [r0])
            p0y = plsc.load_gather(py_v, [r0])
            p0z = plsc.load_gather(pz_v, [r0])
            p1x = plsc.load_gather(px_v, [r1])
            p1y = plsc.load_gather(py_v, [r1])
            p1z = plsc.load_gather(pz_v, [r1])
            jx = pjx - pix
            jy = pjy - piy
            jz = pjz - piz
            ax_ = p0x - pix
            ay_ = p0y - piy
            az_ = p0z - piz
            bx_ = p1x - pix
            by_ = p1y - piy
            bz_ = p1z - piz
            dist2 = jx * jx + jy * jy + jz * jz
            adot = jx * ax_ + jy * ay_ + jz * az_
            crx = jy * az_ - jz * ay_
            cry = jz * ax_ - jx * az_
            crz = jx * ay_ - jy * ax_
            bsq = crx * crx + cry * cry + crz * crz
            p1x_ = ay_ * bz_ - az_ * by_
            p1y_ = az_ * bx_ - ax_ * bz_
            p1z_ = ax_ * by_ - ay_ * bx_
            p2x_ = ay_ * jz - az_ * jy
            p2y_ = az_ * jx - ax_ * jz
            p2z_ = ax_ * jy - ay_ * jx
            a2 = p1x_ * p2x_ + p1y_ * p2y_ + p1z_ * p2z_
            ccx = p1y_ * p2z_ - p1z_ * p2y_
            ccy = p1z_ * p2x_ - p1x_ * p2z_
            ccz = p1x_ * p2y_ - p1y_ * p2x_
            tval = ccx * ax_ + ccy * ay_ + ccz * az_
            r0sq = ax_ * ax_ + ay_ * ay_ + az_ * az_
            dm = (vj - vi).astype(jnp.float32)
            g8_v[0, q, slg] = dist2
            g8_v[1, q, slg] = adot
            g8_v[2, q, slg] = bsq
            g8_v[3, q, slg] = a2
            g8_v[4, q, slg] = tval
            g8_v[5, q, slg] = r0sq
            g8_v[6, q, slg] = dm
            g8_v[7, q, slg] = dm

        row = pl.multiple_of(off // 128, 16)
        pltpu.sync_copy(g8_v, g8_hbm.at[:, pl.ds(row, GCHUNK // 128)])


def _sc_geom(px, py, pz, jidx, iidx):
    mesh = plsc.VectorSubcoreMesh(core_axis_name="c", subcore_axis_name="s",
                                  num_cores=NC, num_subcores=NS)
    f = pl.kernel(
        _geom_body,
        out_type=jax.ShapeDtypeStruct((8, EP // 128, 128), jnp.float32),
        mesh=mesh,
        compiler_params=pltpu.CompilerParams(needs_layout_passes=False),
        scratch_types=[
            pltpu.VMEM((NPAD,), jnp.float32),
            pltpu.VMEM((NPAD,), jnp.float32),
            pltpu.VMEM((NPAD,), jnp.float32),
            pltpu.VMEM((GCHUNK,), jnp.int32),
            pltpu.VMEM((GCHUNK,), jnp.int32),
            pltpu.VMEM((8, GCHUNK // 128, 128), jnp.float32),
        ],
    )
    return f(px, py, pz, jidx, iidx)


RB = EBLK // 128


def _edge_body(g3_ref, wall_ref, act_ref):
    g3 = g3_ref[...]
    dist2 = g3[0]
    a = g3[1]
    bsq = g3[2]
    a2 = g3[3]
    t = g3[4]
    r0sq = g3[5]
    dm = g3[6]

    d = jnp.maximum(jnp.sqrt(dist2), 1e-6)
    dn = a * a + bsq
    ct = jnp.where(dn == 0.0, 1.0, a * lax.rsqrt(dn))
    c2t = 2.0 * ct * ct - 1.0
    b2 = t / (jnp.sqrt(r0sq) + 1e-9)
    qn = a2 * a2 + b2 * b2
    cp = jnp.where(qn == 0.0, 1.0, a2 * lax.rsqrt(qn))
    c2p = 2.0 * cp * cp - 1.0

    x = d * (np.pi / CUTOFF)
    s1 = jnp.sin(x)
    c1 = jnp.cos(x)
    env = jnp.exp(-(d / CUTOFF) ** 2) * np.sqrt(2.0 / CUTOFF) / d
    two_c1 = 2.0 * c1
    sl = [s1, two_c1 * s1]
    for _ in range(4):
        sl.append(two_c1 * sl[-1] - sl[-2])
    rbf = [sn * env for sn in sl]
    angt = [None, ct, c2t]
    angp = [None, cp, c2p]

    rows = []
    for n in range(6):
        for l1 in range(3):
            for l2 in range(3):
                v = rbf[n]
                if angt[l1] is not None:
                    v = v * angt[l1]
                if angp[l2] is not None:
                    v = v * angp[l2]
                rows.append(v)
    for k in range(8):
        fk = float(np.exp(-2.0 * k * np.log(10000.0) / NUM_POS_EMB))
        rows.append(jnp.cos(dm * fk))
    for k in range(8):
        fk = float(np.exp(-2.0 * k * np.log(10000.0) / NUM_POS_EMB))
        rows.append(jnp.sin(dm * fk))

    P = jnp.stack(rows, axis=0).reshape(70, EBLK)
    u = lax.dot_general(P, wall_ref[...],
                        dimension_numbers=(((0,), (0,)), ((), ())),
                        preferred_element_type=jnp.float32)
    act_ref[...] = u * jax.nn.sigmoid(u)


def _edge_act(g3, W_all):
    grid = EP // EBLK
    return pl.pallas_call(
        _edge_body,
        grid=(grid,),
        in_specs=[
            pl.BlockSpec((8, RB, 128), lambda e: (0, e, 0)),
            pl.BlockSpec((70, HID), lambda e: (0, 0)),
        ],
        out_specs=pl.BlockSpec((EBLK, HID), lambda e: (e, 0)),
        out_shape=jax.ShapeDtypeStruct((EP, HID), jnp.float32),
    )(g3, W_all)


def _sc_body(x_hbm, act_hbm, jidx_hbm, iidx_hbm, out_hbm,
             jx, ix, xr, ab, zbuf_v, hacc, gsem, ssem):
    c = lax.axis_index("c")
    s = lax.axis_index("s")

    zeros16 = jnp.zeros((16,), jnp.float32)

    @pl.loop(0, 40)
    def _zero(r):
        for k in range(HID // 16):
            zbuf_v[r, pl.ds(k * 16, 16)] = zeros16

    row0 = s * SLAB

    @pl.loop(0, SLAB // 40)
    def _zslab(r):
        pltpu.sync_copy(zbuf_v, hacc.at[pl.ds(row0 + r * 40, 40)])

    plsc.subcore_barrier()

    w = c * NS + s
    ebase = w * EPT_C

    @pl.loop(0, NCHUNK_C + 4)
    def _pipe(t):
        b = t % 2

        @pl.when(t >= 4)
        def _drain():
            pltpu.make_async_copy(ab.at[b], hacc.at[ix.at[(t - 4) % 4]],
                                  ssem.at[b]).wait()

        @pl.when((t >= 2) & (t < NCHUNK_C + 2))
        def _consume():
            tc = t - 2
            off = pl.multiple_of(ebase + tc * CHUNK, CHUNK)
            pltpu.sync_copy(act_hbm.at[pl.ds(off, CHUNK)], ab.at[b])
            pltpu.make_async_copy(x_hbm.at[jx.at[b]], xr.at[b],
                                  gsem.at[b]).wait()

            @pl.loop(0, CHUNK, unroll=8)
            def _mul(r):
                for k in range(HID // 16):
                    slk = pl.ds(k * 16, 16)
                    ab[b, r, slk] = ab[b, r, slk] * xr[b, r, slk]

            pltpu.async_copy(ab.at[b], hacc.at[ix.at[tc % 4]], ssem.at[b],
                             add=True)

        @pl.when(t < NCHUNK_C)
        def _start():
            off = pl.multiple_of(ebase + t * CHUNK, CHUNK)
            pltpu.sync_copy(jidx_hbm.at[pl.ds(off, CHUNK)], jx.at[b])
            pltpu.sync_copy(iidx_hbm.at[pl.ds(off, CHUNK)], ix.at[t % 4])
            pltpu.async_copy(x_hbm.at[jx.at[b]], xr.at[b], gsem.at[b])

    plsc.subcore_barrier()
    pltpu.sync_copy(hacc.at[pl.ds(row0, SLAB)],
                    out_hbm.at[c, pl.ds(row0, SLAB)])


def _sc_gather_scatter(x, act, jidx, iidx):
    mesh = plsc.VectorSubcoreMesh(core_axis_name="c", subcore_axis_name="s",
                                  num_cores=NC, num_subcores=NS)
    f = pl.kernel(
        _sc_body,
        out_type=jax.ShapeDtypeStruct((NC, SROWS, HID), jnp.float32),
        mesh=mesh,
        scratch_types=[
            pltpu.VMEM((2, CHUNK), jnp.int32),
            pltpu.VMEM((4, CHUNK), jnp.int32),
            pltpu.VMEM((2, CHUNK, HID), jnp.float32),
            pltpu.VMEM((2, CHUNK, HID), jnp.float32),
            pltpu.VMEM((40, HID), jnp.float32),
            pltpu.VMEM_SHARED((SROWS, HID), jnp.float32),
            pltpu.SemaphoreType.DMA((2,)),
            pltpu.SemaphoreType.DMA((2,)),
        ],
    )
    return f(x, act, jidx, iidx)


def _final_body(x_ref, h0_ref, h1_ref, w1_ref, b1_ref, w2_ref, b2_ref,
                wl_ref, bl_ref, wn1_ref, bn1_ref, wno_ref, bno_ref, out_ref):
    h = x_ref[...] + h0_ref[0] + h1_ref[0]
    o = jax.nn.relu(jnp.dot(h, w1_ref[...], preferred_element_type=jnp.float32) + b1_ref[...])
    o = jax.nn.relu(jnp.dot(o, w2_ref[...], preferred_element_type=jnp.float32) + b2_ref[...])
    site = jax.nn.sigmoid(jnp.dot(o, wl_ref[...], preferred_element_type=jnp.float32) + bl_ref[...])
    n1 = jax.nn.relu(jnp.dot(h, wn1_ref[...], preferred_element_type=jnp.float32) + bn1_ref[...])
    node = jnp.dot(n1, wno_ref[...], preferred_element_type=jnp.float32) + bno_ref[...]
    out_ref[...] = jnp.concatenate([site, node], axis=1)


def _final(x, hp, W_out1, b_out1, W_out2, b_out2, W_lin_out, b_lin_out,
           W_node1, b_node1, W_node_out, b_node_out):
    grid = N // NBLK
    return pl.pallas_call(
        _final_body,
        grid=(grid,),
        in_specs=[
            pl.BlockSpec((NBLK, HID), lambda n: (n, 0)),
            pl.BlockSpec((1, NBLK, HID), lambda n: (0, n, 0)),
            pl.BlockSpec((1, NBLK, HID), lambda n: (1, n, 0)),
            pl.BlockSpec((HID, HID), lambda n: (0, 0)),
            pl.BlockSpec((1, HID), lambda n: (0, 0)),
            pl.BlockSpec((HID, 32), lambda n: (0, 0)),
            pl.BlockSpec((1, 32), lambda n: (0, 0)),
            pl.BlockSpec((32, 1), lambda n: (0, 0)),
            pl.BlockSpec((1, 1), lambda n: (0, 0)),
            pl.BlockSpec((HID, 32), lambda n: (0, 0)),
            pl.BlockSpec((1, 32), lambda n: (0, 0)),
            pl.BlockSpec((32, 2), lambda n: (0, 0)),
            pl.BlockSpec((1, 2), lambda n: (0, 0)),
        ],
        out_specs=pl.BlockSpec((NBLK, 3), lambda n: (n, 0)),
        out_shape=jax.ShapeDtypeStruct((N, 3), jnp.float32),
    )(x, hp, hp, W_out1, b_out1, W_out2, b_out2, W_lin_out, b_lin_out,
      W_node1, b_node1, W_node_out, b_node_out)


def kernel(coords_ca, coords_n, coords_c, bb_embs, side_chain_embs, esm_emb,
           W_emb, b_emb, W_esm, b_esm, W_msg, W_pe, W_out1, b_out1, W_out2,
           b_out2, W_lin_out, b_lin_out, W_node1, b_node1, W_node_out,
           b_node_out, z, edge_index, batch):
    del coords_n, coords_c, batch
    npad = NPAD - N
    z2 = jnp.concatenate([z.astype(jnp.int32),
                          jnp.zeros((npad,), jnp.int32)]).reshape(NPAD, 1)
    bb_p = jnp.concatenate([bb_embs, jnp.zeros((npad, 6), jnp.float32)])
    sc_p = jnp.concatenate([side_chain_embs, jnp.zeros((npad, 8), jnp.float32)])
    esm_p = jnp.concatenate([esm_emb, jnp.zeros((npad, 1280), jnp.float32)])
    x = _node_embed(z2, bb_p, sc_p, esm_p,
                    W_emb, b_emb.reshape(1, -1), W_esm,
                    b_esm.reshape(1, -1))

    jidx = jnp.concatenate([edge_index[0].astype(jnp.int32),
                            jnp.zeros((EP - E,), jnp.int32)])
    iidx = jnp.concatenate([edge_index[1].astype(jnp.int32),
                            jnp.full((EP - E,), N, jnp.int32)])
    zpad = jnp.zeros((NPAD - N,), jnp.float32)
    px = jnp.concatenate([coords_ca[:, 0], zpad])
    py = jnp.concatenate([coords_ca[:, 1], zpad])
    pz = jnp.concatenate([coords_ca[:, 2], zpad])
    g8 = _sc_geom(px, py, pz, jidx, iidx)
    W_all = jnp.concatenate([W_msg, W_pe], axis=0)
    act = _edge_act(g8, W_all)
    hp = _sc_gather_scatter(x, act, jidx, iidx)
    return _final(x, hp, W_out1, b_out1.reshape(1, -1), W_out2,
                  b_out2.reshape(1, -1), W_lin_out, b_lin_out.reshape(1, -1),
                  W_node1, b_node1.reshape(1, -1), W_node_out,
                  b_node_out.reshape(1, -1))


# speedup vs baseline: 1.0449x; 1.0005x over previous
import functools

import numpy as np
import jax
import jax.numpy as jnp
from jax import lax
from jax.experimental import pallas as pl
from jax.experimental.pallas import tpu as pltpu
from jax.experimental.pallas import tpu_sc as plsc

N = 10000
E = 320000
CUTOFF = 11.5
NUM_RADIAL = 6
NUM_SPH = 3
NUM_POS_EMB = 16
HID = 128

NC = 2
NS = 16
NW = NC * NS
EP = 327680
EDGES_PER_TILE = EP // NW
CHUNK = 80
EPT_C = EP // NW
NCHUNK_C = EPT_C // CHUNK
GCHUNK = 2048
NGCHUNK = EDGES_PER_TILE // GCHUNK
NPAD = 10240
SROWS = 10240
SLAB = SROWS // NS

NBLK = 1000
NBLK_A = 640
EBLK = 2048


def _embed_body(z_ref, bb_ref, sc_ref, esm_ref, wemb_ref, bemb_ref,
                wesm_ref, besm_ref, x_ref):
    z = z_ref[...]
    onehot = (lax.broadcasted_iota(jnp.int32, (NBLK_A, 26), 1) == z).astype(jnp.float32)
    xin = jnp.concatenate([onehot, bb_ref[...], sc_ref[...]], axis=1)
    lo = jnp.dot(xin, wemb_ref[...], preferred_element_type=jnp.float32) + bemb_ref[...]
    hi = jnp.dot(esm_ref[...], wesm_ref[...], preferred_element_type=jnp.float32) + besm_ref[...]
    x_ref[...] = jnp.concatenate([lo, hi], axis=1)


def _node_embed(z, bb, sc, esm, W_emb, b_emb, W_esm, b_esm):
    grid = NPAD // NBLK_A
    return pl.pallas_call(
        _embed_body,
        grid=(grid,),
        in_specs=[
            pl.BlockSpec((NBLK_A, 1), lambda n: (n, 0)),
            pl.BlockSpec((NBLK_A, 6), lambda n: (n, 0)),
            pl.BlockSpec((NBLK_A, 8), lambda n: (n, 0)),
            pl.BlockSpec((NBLK_A, 1280), lambda n: (n, 0)),
            pl.BlockSpec((40, 64), lambda n: (0, 0)),
            pl.BlockSpec((1, 64), lambda n: (0, 0)),
            pl.BlockSpec((1280, 64), lambda n: (0, 0)),
            pl.BlockSpec((1, 64), lambda n: (0, 0)),
        ],
        out_specs=pl.BlockSpec((NBLK_A, HID), lambda n: (n, 0)),
        out_shape=jax.ShapeDtypeStruct((NPAD, HID), jnp.float32),
    )(z, bb, sc, esm, W_emb, b_emb, W_esm, b_esm)


def _geom_body(px_hbm, py_hbm, pz_hbm, jidx_hbm, iidx_hbm, g8_hbm,
               px_v, py_v, pz_v, idxj_v, idxi_v, g8_v):
    c = lax.axis_index("c")
    s = lax.axis_index("s")
    pltpu.sync_copy(px_hbm, px_v)
    pltpu.sync_copy(py_hbm, py_v)
    pltpu.sync_copy(pz_hbm, pz_v)
    base = (c * NS + s) * EDGES_PER_TILE

    @pl.loop(0, NGCHUNK)
    def _chunk(t):
        off = base + t * GCHUNK
        pltpu.sync_copy(jidx_hbm.at[pl.ds(off, GCHUNK)], idxj_v)
        pltpu.sync_copy(iidx_hbm.at[pl.ds(off, GCHUNK)], idxi_v)

        @pl.loop(0, GCHUNK // 16)
        def _group(gr):
            sl = pl.ds(gr * 16, 16)
            q = gr // 8
            lo = (gr % 8) * 16
            slg = pl.ds(lo, 16)
            vi = idxi_v[sl]
            vj = idxj_v[sl]
            r0 = jnp.where(vi == 0, N - 1, vi - 1)
            r1 = jnp.where(vi == N - 1, 0, vi + 1)
            pjx = plsc.load_gather(px_v, [vj])
            pjy = plsc.load_gather(py_v, [vj])
            pjz = plsc.load_gather(pz_v, [vj])
            pix = plsc.load_gather(px_v, [vi])
            piy = plsc.load_gather(py_v, [vi])
            piz = plsc.load_gather(pz_v, [vi])
            p0x = plsc.load_gather(px_v, [r0])
            p0y = plsc.load_gather(py_v, [r0])
            p0z = plsc.load_gather(pz_v, [r0])
            p1x = plsc.load_gather(px_v, [r1])
            p1y = plsc.load_gather(py_v, [r1])
            p1z = plsc.load_gather(pz_v, [r1])
            jx = pjx - pix
            jy = pjy - piy
            jz = pjz - piz
            ax_ = p0x - pix
            ay_ = p0y - piy
            az_ = p0z - piz
            bx_ = p1x - pix
            by_ = p1y - piy
            bz_ = p1z - piz
            dist2 = jx * jx + jy * jy + jz * jz
            adot = jx * ax_ + jy * ay_ + jz * az_
            crx = jy * az_ - jz * ay_
            cry = jz * ax_ - jx * az_
            crz = jx * ay_ - jy * ax_
            bsq = crx * crx + cry * cry + crz * crz
            p1x_ = ay_ * bz_ - az_ * by_
            p1y_ = az_ * bx_ - ax_ * bz_
            p1z_ = ax_ * by_ - ay_ * bx_
            p2x_ = ay_ * jz - az_ * jy
            p2y_ = az_ * jx - ax_ * jz
            p2z_ = ax_ * jy - ay_ * jx
            a2 = p1x_ * p2x_ + p1y_ * p2y_ + p1z_ * p2z_
            ccx = p1y_ * p2z_ - p1z_ * p2y_
            ccy = p1z_ * p2x_ - p1x_ * p2z_
            ccz = p1x_ * p2y_ - p1y_ * p2x_
            tval = ccx * ax_ + ccy * ay_ + ccz * az_
            r0sq = ax_ * ax_ + ay_ * ay_ + az_ * az_
            dm = (vj - vi).astype(jnp.float32)
            g8_v[0, q, slg] = dist2
            g8_v[1, q, slg] = adot
            g8_v[2, q, slg] = bsq
            g8_v[3, q, slg] = a2
            g8_v[4, q, slg] = tval
            g8_v[5, q, slg] = r0sq
            g8_v[6, q, slg] = dm
            g8_v[7, q, slg] = dm

        row = pl.multiple_of(off // 128, 16)
        pltpu.sync_copy(g8_v, g8_hbm.at[:, pl.ds(row, GCHUNK // 128)])


def _sc_geom(px, py, pz, jidx, iidx):
    mesh = plsc.VectorSubcoreMesh(core_axis_name="c", subcore_axis_name="s",
                                  num_cores=NC, num_subcores=NS)
    f = pl.kernel(
        _geom_body,
        out_type=jax.ShapeDtypeStruct((8, EP // 128, 128), jnp.float32),
        mesh=mesh,
        compiler_params=pltpu.CompilerParams(needs_layout_passes=False),
        scratch_types=[
            pltpu.VMEM((NPAD,), jnp.float32),
            pltpu.VMEM((NPAD,), jnp.float32),
            pltpu.VMEM((NPAD,), jnp.float32),
            pltpu.VMEM((GCHUNK,), jnp.int32),
            pltpu.VMEM((GCHUNK,), jnp.int32),
            pltpu.VMEM((8, GCHUNK // 128, 128), jnp.float32),
        ],
    )
    return f(px, py, pz, jidx, iidx)


RB = EBLK // 128


def _edge_body(g3_ref, wall_ref, act_ref):
    g3 = g3_ref[...]
    dist2 = g3[0]
    a = g3[1]
    bsq = g3[2]
    a2 = g3[3]
    t = g3[4]
    r0sq = g3[5]
    dm = g3[6]

    d = jnp.maximum(jnp.sqrt(dist2), 1e-6)
    dn = a * a + bsq
    ct = jnp.where(dn == 0.0, 1.0, a * lax.rsqrt(dn))
    c2t = 2.0 * ct * ct - 1.0
    b2 = t / (jnp.sqrt(r0sq) + 1e-9)
    qn = a2 * a2 + b2 * b2
    cp = jnp.where(qn == 0.0, 1.0, a2 * lax.rsqrt(qn))
    c2p = 2.0 * cp * cp - 1.0

    x = d * (np.pi / CUTOFF)
    s1 = jnp.sin(x)
    c1 = jnp.cos(x)
    env = jnp.exp(-(d / CUTOFF) ** 2) * np.sqrt(2.0 / CUTOFF) / d
    two_c1 = 2.0 * c1
    sl = [s1, two_c1 * s1]
    for _ in range(4):
        sl.append(two_c1 * sl[-1] - sl[-2])
    rbf = [sn * env for sn in sl]
    angt = [None, ct, c2t]
    angp = [None, cp, c2p]

    rows = []
    for n in range(6):
        for l1 in range(3):
            for l2 in range(3):
                v = rbf[n]
                if angt[l1] is not None:
                    v = v * angt[l1]
                if angp[l2] is not None:
                    v = v * angp[l2]
                rows.append(v)
    for k in range(8):
        fk = float(np.exp(-2.0 * k * np.log(10000.0) / NUM_POS_EMB))
        rows.append(jnp.cos(dm * fk))
    for k in range(8):
        fk = float(np.exp(-2.0 * k * np.log(10000.0) / NUM_POS_EMB))
        rows.append(jnp.sin(dm * fk))

    P = jnp.stack(rows, axis=0).reshape(70, EBLK)
    u = lax.dot_general(P, wall_ref[...],
                        dimension_numbers=(((0,), (0,)), ((), ())),
                        preferred_element_type=jnp.float32)
    act_ref[...] = u * jax.nn.sigmoid(u)


def _edge_act(g3, W_all):
    grid = EP // EBLK
    return pl.pallas_call(
        _edge_body,
        grid=(grid,),
        in_specs=[
            pl.BlockSpec((8, RB, 128), lambda e: (0, e, 0)),
            pl.BlockSpec((70, HID), lambda e: (0, 0)),
        ],
        out_specs=pl.BlockSpec((EBLK, HID), lambda e: (e, 0)),
        out_shape=jax.ShapeDtypeStruct((EP, HID), jnp.float32),
    )(g3, W_all)


def _sc_body(x_hbm, act_hbm, jidx_hbm, iidx_hbm, out_hbm,
             jx, ix, xr, ab, zbuf_v, hacc, gsem, ssem):
    c = lax.axis_index("c")
    s = lax.axis_index("s")

    zeros16 = jnp.zeros((16,), jnp.float32)

    @pl.loop(0, 40)
    def _zero(r):
        for k in range(HID // 16):
            zbuf_v[r, pl.ds(k * 16, 16)] = zeros16

    row0 = s * SLAB

    @pl.loop(0, SLAB // 40)
    def _zslab(r):
        pltpu.sync_copy(zbuf_v, hacc.at[pl.ds(row0 + r * 40, 40)])

    plsc.subcore_barrier()

    w = c * NS + s
    ebase = w * EPT_C

    @pl.loop(0, NCHUNK_C + 4)
    def _pipe(t):
        b = t % 2

        @pl.when(t >= 4)
        def _drain():
            pltpu.make_async_copy(ab.at[b], hacc.at[ix.at[(t - 4) % 4]],
                                  ssem.at[b]).wait()

        @pl.when((t >= 2) & (t < NCHUNK_C + 2))
        def _consume():
            tc = t - 2
            off = pl.multiple_of(ebase + tc * CHUNK, CHUNK)
            pltpu.sync_copy(act_hbm.at[pl.ds(off, CHUNK)], ab.at[b])
            pltpu.make_async_copy(x_hbm.at[jx.at[b]], xr.at[b],
                                  gsem.at[b]).wait()

            @pl.loop(0, CHUNK, unroll=8)
            def _mul(r):
                for k in range(HID // 16):
                    slk = pl.ds(k * 16, 16)
                    ab[b, r, slk] = ab[b, r, slk] * xr[b, r, slk]

            pltpu.async_copy(ab.at[b], hacc.at[ix.at[tc % 4]], ssem.at[b],
                             add=True)

        @pl.when(t < NCHUNK_C)
        def _start():
            off = pl.multiple_of(ebase + t * CHUNK, CHUNK)
            pltpu.sync_copy(jidx_hbm.at[pl.ds(off, CHUNK)], jx.at[b])
            pltpu.sync_copy(iidx_hbm.at[pl.ds(off, CHUNK)], ix.at[t % 4])
            pltpu.async_copy(x_hbm.at[jx.at[b]], xr.at[b], gsem.at[b])

    plsc.subcore_barrier()
    pltpu.sync_copy(hacc.at[pl.ds(row0, SLAB)],
                    out_hbm.at[c, pl.ds(row0, SLAB)])


def _sc_gather_scatter(x, act, jidx, iidx):
    mesh = plsc.VectorSubcoreMesh(core_axis_name="c", subcore_axis_name="s",
                                  num_cores=NC, num_subcores=NS)
    f = pl.kernel(
        _sc_body,
        out_type=jax.ShapeDtypeStruct((NC, SROWS, HID), jnp.float32),
        mesh=mesh,
        scratch_types=[
            pltpu.VMEM((2, CHUNK), jnp.int32),
            pltpu.VMEM((4, CHUNK), jnp.int32),
            pltpu.VMEM((2, CHUNK, HID), jnp.float32),
            pltpu.VMEM((2, CHUNK, HID), jnp.float32),
            pltpu.VMEM((40, HID), jnp.float32),
            pltpu.VMEM_SHARED((SROWS, HID), jnp.float32),
            pltpu.SemaphoreType.DMA((2,)),
            pltpu.SemaphoreType.DMA((2,)),
        ],
    )
    return f(x, act, jidx, iidx)


def _final_body(x_ref, h0_ref, h1_ref, w1_ref, b1_ref, w2_ref, b2_ref,
                wl_ref, bl_ref, wn1_ref, bn1_ref, wno_ref, bno_ref, out_ref):
    h = x_ref[...] + h0_ref[0] + h1_ref[0]
    o = jax.nn.relu(jnp.dot(h, w1_ref[...], preferred_element_type=jnp.float32) + b1_ref[...])
    o = jax.nn.relu(jnp.dot(o, w2_ref[...], preferred_element_type=jnp.float32) + b2_ref[...])
    site = jax.nn.sigmoid(jnp.dot(o, wl_ref[...], preferred_element_type=jnp.float32) + bl_ref[...])
    n1 = jax.nn.relu(jnp.dot(h, wn1_ref[...], preferred_element_type=jnp.float32) + bn1_ref[...])
    node = jnp.dot(n1, wno_ref[...], preferred_element_type=jnp.float32) + bno_ref[...]
    out_ref[...] = jnp.concatenate([site, node], axis=1)


def _final(x, hp, W_out1, b_out1, W_out2, b_out2, W_lin_out, b_lin_out,
           W_node1, b_node1, W_node_out, b_node_out):
    grid = N // NBLK
    return pl.pallas_call(
        _final_body,
        grid=(grid,),
        in_specs=[
            pl.BlockSpec((NBLK, HID), lambda n: (n, 0)),
            pl.BlockSpec((1, NBLK, HID), lambda n: (0, n, 0)),
            pl.BlockSpec((1, NBLK, HID), lambda n: (1, n, 0)),
            pl.BlockSpec((HID, HID), lambda n: (0, 0)),
            pl.BlockSpec((1, HID), lambda n: (0, 0)),
            pl.BlockSpec((HID, 32), lambda n: (0, 0)),
            pl.BlockSpec((1, 32), lambda n: (0, 0)),
            pl.BlockSpec((32, 1), lambda n: (0, 0)),
            pl.BlockSpec((1, 1), lambda n: (0, 0)),
            pl.BlockSpec((HID, 32), lambda n: (0, 0)),
            pl.BlockSpec((1, 32), lambda n: (0, 0)),
            pl.BlockSpec((32, 2), lambda n: (0, 0)),
            pl.BlockSpec((1, 2), lambda n: (0, 0)),
        ],
        out_specs=pl.BlockSpec((NBLK, 3), lambda n: (n, 0)),
        out_shape=jax.ShapeDtypeStruct((N, 3), jnp.float32),
    )(x, hp, hp, W_out1, b_out1, W_out2, b_out2, W_lin_out, b_lin_out,
      W_node1, b_node1, W_node_out, b_node_out)


def kernel(coords_ca, coords_n, coords_c, bb_embs, side_chain_embs, esm_emb,
           W_emb, b_emb, W_esm, b_esm, W_msg, W_pe, W_out1, b_out1, W_out2,
           b_out2, W_lin_out, b_lin_out, W_node1, b_node1, W_node_out,
           b_node_out, z, edge_index, batch):
    del coords_n, coords_c, batch
    npad = NPAD - N
    z2 = jnp.concatenate([z.astype(jnp.int32),
                          jnp.zeros((npad,), jnp.int32)]).reshape(NPAD, 1)
    bb_p = jnp.concatenate([bb_embs, jnp.zeros((npad, 6), jnp.float32)])
    sc_p = jnp.concatenate([side_chain_embs, jnp.zeros((npad, 8), jnp.float32)])
    esm_p = jnp.concatenate([esm_emb, jnp.zeros((npad, 1280), jnp.float32)])
    x = _node_embed(z2, bb_p, sc_p, esm_p,
                    W_emb, b_emb.reshape(1, -1), W_esm,
                    b_esm.reshape(1, -1))

    jidx = jnp.concatenate([edge_index[0].astype(jnp.int32),
                            jnp.zeros((EP - E,), jnp.int32)])
    iidx = jnp.concatenate([edge_index[1].astype(jnp.int32),
                            jnp.full((EP - E,), N, jnp.int32)])
    zpad = jnp.zeros((NPAD - N,), jnp.float32)
    px = jnp.concatenate([coords_ca[:, 0], zpad])
    py = jnp.concatenate([coords_ca[:, 1], zpad])
    pz = jnp.concatenate([coords_ca[:, 2], zpad])
    g8 = _sc_geom(px, py, pz, jidx, iidx)
    W_all = jnp.concatenate([W_msg, W_pe], axis=0)
    act = _edge_act(g8, W_all)
    hp = _sc_gather_scatter(x, act, jidx, iidx)
    return _final(x, hp, W_out1, b_out1.reshape(1, -1), W_out2,
                  b_out2.reshape(1, -1), W_lin_out, b_lin_out.reshape(1, -1),
                  W_node1, b_node1.reshape(1, -1), W_node_out,
                  b_node_out.reshape(1, -1))
